# baseline (speedup 1.0000x reference)
import functools
import math

import jax
import jax.numpy as jnp
from jax import lax
from jax.experimental import pallas as pl
from jax.experimental.pallas import tpu as pltpu
from jax.experimental.pallas import tpu_sc as plsc

_EPS = 1e-8
_DIM = 64
_B = 16384
_NBIG = 13
_NSMALL = 7
_NW = 32
_CHUNK = 512
_SUB = 128
_CB = 512
_INV_MEAN = 1.0 / (_B * _DIM)


def _sc_gather(min_e, delta_e, rel_e, scal_e, idx_big, idx_small):
    nbig = _NBIG * _B
    nsmall = _NSMALL * _B
    big_pw = nbig // _NW
    small_pw = nsmall // _NW
    mesh = plsc.VectorSubcoreMesh(core_axis_name="c", subcore_axis_name="s")

    @functools.partial(
        pl.kernel,
        mesh=mesh,
        compiler_params=pltpu.CompilerParams(use_tc_tiling_on_sc=False),
        out_type=[
            jax.ShapeDtypeStruct((nbig, _DIM), jnp.float32),
            jax.ShapeDtypeStruct((nbig, _DIM), jnp.float32),
            jax.ShapeDtypeStruct((nsmall, _DIM), jnp.float32),
            jax.ShapeDtypeStruct((nsmall, _DIM), jnp.float32),
        ],
        scratch_types=[
            pltpu.VMEM((_CHUNK,), jnp.int32),
            pltpu.VMEM((_CHUNK, _DIM), jnp.float32),
            pltpu.VMEM((_CHUNK, _DIM), jnp.float32),
            pltpu.SemaphoreType.DMA,
            pltpu.SemaphoreType.DMA,
        ],
    )
    def gather_kernel(min_hbm, dl_hbm, rel_hbm, scal_hbm, ib_hbm, is_hbm,
                      omn_hbm, odl_hbm, orel_hbm, oscal_hbm,
                      idx_v, buf_a, buf_b, sem_a, sem_b):
        wid = lax.axis_index("s") * 2 + lax.axis_index("c")

        def run(tab1, tab2, out1, out2, idx_hbm, base, nchunks):
            @pl.loop(0, nchunks)
            def _(ci):
                off = base + ci * _CHUNK
                pltpu.sync_copy(idx_hbm.at[pl.ds(off, _CHUNK)], idx_v)
                waits = []
                for s in range(_CHUNK // _SUB):
                    sl = pl.ds(s * _SUB, _SUB)
                    waits.append(pltpu.async_copy(
                        tab1.at[idx_v.at[sl]], buf_a.at[sl], sem_a))
                    waits.append(pltpu.async_copy(
                        tab2.at[idx_v.at[sl]], buf_b.at[sl], sem_b))
                for w in waits:
                    w.wait()
                pltpu.sync_copy(buf_a, out1.at[pl.ds(off, _CHUNK)])
                pltpu.sync_copy(buf_b, out2.at[pl.ds(off, _CHUNK)])

        run(min_hbm, dl_hbm, omn_hbm, odl_hbm, ib_hbm,
            wid * big_pw, big_pw // _CHUNK)
        run(rel_hbm, scal_hbm, orel_hbm, oscal_hbm, is_hbm,
            wid * small_pw, small_pw // _CHUNK)

    return gather_kernel(min_e, delta_e, rel_e, scal_e, idx_big, idx_small)


def _vol_clip(diff):
    sp = jnp.maximum(diff, 0.0) + jnp.log1p(jnp.exp(-jnp.abs(diff)))
    n = sp.shape[-1]
    while n > 1:
        n //= 2
        sp = sp[:, :n] * sp[:, n:]
    return jnp.clip(sp[:, 0], 1e-10, 1e4)


def _reg_sum(mn, mx):
    return (jnp.sum(jnp.maximum(mx - 1.0 + _EPS, 0.0))
            + jnp.sum(jnp.maximum(-mn - _EPS, 0.0)))


def _tc_body(mn_ref, dl_ref, rel_ref, scal_ref, *outs):
    i = pl.program_id(0)

    @pl.when(i == 0)
    def _():
        for o in outs:
            o[0, 0] = 0.0

    mn_all = mn_ref[...]
    ex_all = jnp.exp(dl_ref[...])
    mx_all = mn_all + ex_all
    rel_all = rel_ref[...]
    scal_all = scal_ref[...]

    def box(s):
        return mn_all[s], mx_all[s]

    def inter_diff(mn1, mx1, mn2, mx2):
        return jnp.minimum(mx1, mx2) - jnp.maximum(mn1, mn2)

    mn0, mx0 = box(0)
    mn1, mx1 = box(1)
    nf1_loss = jnp.sum(
        1.0 - _vol_clip(inter_diff(mn0, mx0, mn1, mx1)) / _vol_clip(ex_all[0]))
    nf1_reg = (_reg_sum(mn0, mx0) + _reg_sum(mn1, mx1)) * _INV_MEAN

    mn2, mx2 = box(2)
    mn3, mx3 = box(3)
    mn4, mx4 = box(4)
    imn = jnp.maximum(mn2, mn3)
    imx = jnp.minimum(mx2, mx3)
    nf2_loss = jnp.sum(
        1.0 - _vol_clip(inter_diff(imn, imx, mn4, mx4)) / _vol_clip(imx - imn))
    nf2_reg = (_reg_sum(imn, imx) + _reg_sum(mn2, mx2)
               + _reg_sum(mn3, mx3) + _reg_sum(mn4, mx4)) * _INV_MEAN

    mn5, mx5 = box(5)
    mn6, mx6 = box(6)
    scp = scal_all[0] + _EPS
    rel = rel_all[0]
    tmn = mn5 * scp + rel
    tmx = mx5 * scp + rel
    nf3_loss = jnp.sum(
        1.0 - _vol_clip(inter_diff(tmn, tmx, mn6, mx6))
        / _vol_clip(ex_all[5] * scp))
    nf3_reg = (_reg_sum(tmn, tmx) + _reg_sum(mn5, mx5)
               + _reg_sum(mn6, mx6)) * _INV_MEAN

    mn7, mx7 = box(7)
    mn8, mx8 = box(8)
    scp = scal_all[1] + _EPS
    rel = rel_all[1]
    tmn = (mn7 - rel) / scp
    tmx = (mx7 - rel) / scp
    nf4_loss = jnp.sum(
        1.0 - _vol_clip(inter_diff(tmn, tmx, mn8, mx8))
        / _vol_clip(ex_all[7] / scp))
    nf4_reg = (_reg_sum(tmn, tmx) + _reg_sum(mn7, mx7)
               + _reg_sum(mn8, mx8)) * _INV_MEAN

    mn9, mx9 = box(9)
    mn10, mx10 = box(10)
    dis_loss = jnp.sum(
        _vol_clip(inter_diff(mn9, mx9, mn10, mx10))
        / (_vol_clip(ex_all[9]) * _vol_clip(ex_all[10])))
    dis_reg = (_reg_sum(mn9, mx9) + _reg_sum(mn10, mx10)) * _INV_MEAN

    mn11, mx11 = box(11)
    mn12, mx12 = box(12)
    nf1n_loss = jnp.sum(
        _vol_clip(inter_diff(mn11, mx11, mn12, mx12)) / _vol_clip(ex_all[11]))
    nf1n_reg = (_reg_sum(mn11, mx11) + _reg_sum(mn12, mx12)) * _INV_MEAN

    t1, t2 = rel_all[2], rel_all[3]
    s1, s2 = scal_all[2], scal_all[3]
    n1 = jnp.sqrt(jnp.sum(jnp.maximum(t1 - t2, 0.0) ** 2, axis=1))
    n2 = jnp.sqrt(jnp.sum(
        jnp.maximum(s1 / (s2 + _EPS) - 1.0, 0.0) ** 2, axis=1))
    role_inc = jnp.sum(n1 + n2)

    t1, t2, t3 = rel_all[4], rel_all[5], rel_all[6]
    s1, s2, s3 = scal_all[4], scal_all[5], scal_all[6]
    n1 = jnp.sqrt(jnp.sum(jnp.maximum(t1 + t2 - t3, 0.0) ** 2, axis=1))
    n2 = jnp.sqrt(jnp.sum(
        jnp.maximum(s1 * s2 / (s3 + _EPS) - 1.0, 0.0) ** 2, axis=1))
    role_chain = jnp.sum(n1 + n2)

    vals = (nf1_loss, nf1n_loss, nf2_loss, nf3_loss, nf4_loss, dis_loss,
            role_inc, role_chain,
            nf1_reg, nf1n_reg, nf2_reg, nf3_reg, nf4_reg, dis_reg)
    for o, v in zip(outs, vals):
        o[0, 0] += v


def _tc_compute(big_mn, big_dl, rel_rows, scal_rows):
    grid = (_B // _CB,)
    scalar_spec = pl.BlockSpec((1, 1), lambda i: (0, 0),
                               memory_space=pltpu.SMEM)
    return pl.pallas_call(
        _tc_body,
        grid=grid,
        in_specs=[
            pl.BlockSpec((_NBIG, _CB, _DIM), lambda i: (0, i, 0)),
            pl.BlockSpec((_NBIG, _CB, _DIM), lambda i: (0, i, 0)),
            pl.BlockSpec((_NSMALL, _CB, _DIM), lambda i: (0, i, 0)),
            pl.BlockSpec((_NSMALL, _CB, _DIM), lambda i: (0, i, 0)),
        ],
        out_specs=[scalar_spec] * 14,
        out_shape=[jax.ShapeDtypeStruct((1, 1), jnp.float32)] * 14,
    )(big_mn, big_dl, rel_rows, scal_rows)


def kernel(min_embedding, delta_embedding, relation_embedding,
           scaling_embedding, data0, data1, data2, data3, data4, data5,
           data6, data7):
    d0 = data0.astype(jnp.int32)
    d1 = data1.astype(jnp.int32)
    d2 = data2.astype(jnp.int32)
    d3 = data3.astype(jnp.int32)
    d4 = data4.astype(jnp.int32)
    d5 = data5.astype(jnp.int32)
    d6 = data6.astype(jnp.int32)
    d7 = data7.astype(jnp.int32)
    idx_big = jnp.concatenate([
        d0[:, 0], d0[:, 2],
        d1[:, 0], d1[:, 1], d1[:, 2],
        d2[:, 0], d2[:, 2],
        d3[:, 1], d3[:, 2],
        d4[:, 0], d4[:, 1],
        d5[:, 0], d5[:, 2],
    ])
    idx_small = jnp.concatenate([
        d2[:, 1], d3[:, 0],
        d6[:, 0], d6[:, 1],
        d7[:, 0], d7[:, 1], d7[:, 2],
    ])
    big_mn, big_dl, rel_rows, scal_rows = _sc_gather(
        min_embedding, delta_embedding, relation_embedding,
        scaling_embedding, idx_big, idx_small)
    outs = _tc_compute(
        big_mn.reshape(_NBIG, _B, _DIM),
        big_dl.reshape(_NBIG, _B, _DIM),
        rel_rows.reshape(_NSMALL, _B, _DIM),
        scal_rows.reshape(_NSMALL, _B, _DIM))
    return tuple(o.reshape(()) for o in outs)


# speedup vs baseline: 1.7826x; 1.7826x over previous
import functools
import math

import jax
import jax.numpy as jnp
from jax import lax
from jax.experimental import pallas as pl
from jax.experimental.pallas import tpu as pltpu
from jax.experimental.pallas import tpu_sc as plsc

_EPS = 1e-8
_DIM = 64
_B = 16384
_NBIG = 13
_NSMALL = 7
_NW = 32
_WPB = _B // _NW
_SUB = 128
_CB = 512
_INV_MEAN = 1.0 / (_B * _DIM)
_LOG_LO = math.log(1e-10)
_LOG_HI = math.log(1e4)


def _sc_gather(concept_tab, relation_tab, idx_big, idx_small):
    mesh = plsc.VectorSubcoreMesh(core_axis_name="c", subcore_axis_name="s")

    @functools.partial(
        pl.kernel,
        mesh=mesh,
        compiler_params=pltpu.CompilerParams(use_tc_tiling_on_sc=False),
        out_type=[
            jax.ShapeDtypeStruct((_NBIG, _B, 2 * _DIM), jnp.float32),
            jax.ShapeDtypeStruct((_NSMALL, _B, 2 * _DIM), jnp.float32),
        ],
        scratch_types=[
            pltpu.VMEM((_WPB,), jnp.int32),
            pltpu.VMEM((_WPB, 2 * _DIM), jnp.float32),
            pltpu.SemaphoreType.DMA,
        ],
    )
    def gather_kernel(tab_hbm, rel_hbm, ib_hbm, is_hbm, obig_hbm, osmall_hbm,
                      idx_v, buf, sem):
        wid = lax.axis_index("s") * 2 + lax.axis_index("c")
        base = wid * _WPB

        def run(tab, out, idx_hbm, nseg):
            for seg in range(nseg):
                pltpu.sync_copy(idx_hbm.at[seg, pl.ds(base, _WPB)], idx_v)
                waits = []
                for s in range(_WPB // _SUB):
                    sl = pl.ds(s * _SUB, _SUB)
                    waits.append(pltpu.async_copy(
                        tab.at[idx_v.at[sl]], buf.at[sl], sem))
                for w in waits:
                    w.wait()
                pltpu.sync_copy(buf, out.at[seg, pl.ds(base, _WPB)])

        run(tab_hbm, obig_hbm, ib_hbm, _NBIG)
        run(rel_hbm, osmall_hbm, is_hbm, _NSMALL)

    return gather_kernel(concept_tab, relation_tab, idx_big, idx_small)


def _lv(diff):
    sp = jnp.maximum(diff, 0.0) + jnp.log1p(jnp.exp(-jnp.abs(diff)))
    return jnp.clip(jnp.sum(jnp.log(sp), axis=1), _LOG_LO, _LOG_HI)


def _reg_sum(mn, mx):
    return (jnp.sum(jnp.maximum(mx - 1.0 + _EPS, 0.0))
            + jnp.sum(jnp.maximum(-mn - _EPS, 0.0)))


def _tc_body(big_ref, small_ref, *outs):
    i = pl.program_id(0)

    @pl.when(i == 0)
    def _():
        for o in outs:
            o[0, 0] = 0.0

    def box(s):
        row = big_ref[s]
        mn = row[:, :_DIM]
        ex = jnp.exp(row[:, _DIM:])
        return mn, mn + ex, ex

    def rels(s):
        row = small_ref[s]
        return row[:, :_DIM], row[:, _DIM:]

    def inter_diff(mn1, mx1, mn2, mx2):
        return jnp.minimum(mx1, mx2) - jnp.maximum(mn1, mn2)

    mn0, mx0, ex0 = box(0)
    mn1, mx1, _ = box(1)
    nf1_loss = jnp.sum(
        1.0 - jnp.exp(_lv(inter_diff(mn0, mx0, mn1, mx1)) - _lv(ex0)))
    nf1_reg = (_reg_sum(mn0, mx0) + _reg_sum(mn1, mx1)) * _INV_MEAN

    mn2, mx2, _ = box(2)
    mn3, mx3, _ = box(3)
    mn4, mx4, _ = box(4)
    imn = jnp.maximum(mn2, mn3)
    imx = jnp.minimum(mx2, mx3)
    nf2_loss = jnp.sum(
        1.0 - jnp.exp(_lv(inter_diff(imn, imx, mn4, mx4)) - _lv(imx - imn)))
    nf2_reg = (_reg_sum(imn, imx) + _reg_sum(mn2, mx2)
               + _reg_sum(mn3, mx3) + _reg_sum(mn4, mx4)) * _INV_MEAN

    mn5, mx5, ex5 = box(5)
    mn6, mx6, _ = box(6)
    rel, sc = rels(0)
    scp = sc + _EPS
    tmn = mn5 * scp + rel
    tmx = mx5 * scp + rel
    nf3_loss = jnp.sum(
        1.0 - jnp.exp(_lv(inter_diff(tmn, tmx, mn6, mx6)) - _lv(ex5 * scp)))
    nf3_reg = (_reg_sum(tmn, tmx) + _reg_sum(mn5, mx5)
               + _reg_sum(mn6, mx6)) * _INV_MEAN

    mn7, mx7, ex7 = box(7)
    mn8, mx8, _ = box(8)
    rel, sc = rels(1)
    scp = sc + _EPS
    tmn = (mn7 - rel) / scp
    tmx = (mx7 - rel) / scp
    nf4_loss = jnp.sum(
        1.0 - jnp.exp(_lv(inter_diff(tmn, tmx, mn8, mx8)) - _lv(ex7 / scp)))
    nf4_reg = (_reg_sum(tmn, tmx) + _reg_sum(mn7, mx7)
               + _reg_sum(mn8, mx8)) * _INV_MEAN

    mn9, mx9, ex9 = box(9)
    mn10, mx10, ex10 = box(10)
    dis_loss = jnp.sum(jnp.exp(
        _lv(inter_diff(mn9, mx9, mn10, mx10)) - (_lv(ex9) + _lv(ex10))))
    dis_reg = (_reg_sum(mn9, mx9) + _reg_sum(mn10, mx10)) * _INV_MEAN

    mn11, mx11, ex11 = box(11)
    mn12, mx12, _ = box(12)
    nf1n_loss = jnp.sum(jnp.exp(
        _lv(inter_diff(mn11, mx11, mn12, mx12)) - _lv(ex11)))
    nf1n_reg = (_reg_sum(mn11, mx11) + _reg_sum(mn12, mx12)) * _INV_MEAN

    t1, s1 = rels(2)
    t2, s2 = rels(3)
    n1 = jnp.sqrt(jnp.sum(jnp.maximum(t1 - t2, 0.0) ** 2, axis=1))
    n2 = jnp.sqrt(jnp.sum(
        jnp.maximum(s1 / (s2 + _EPS) - 1.0, 0.0) ** 2, axis=1))
    role_inc = jnp.sum(n1 + n2)

    t1, s1 = rels(4)
    t2, s2 = rels(5)
    t3, s3 = rels(6)
    n1 = jnp.sqrt(jnp.sum(jnp.maximum(t1 + t2 - t3, 0.0) ** 2, axis=1))
    n2 = jnp.sqrt(jnp.sum(
        jnp.maximum(s1 * s2 / (s3 + _EPS) - 1.0, 0.0) ** 2, axis=1))
    role_chain = jnp.sum(n1 + n2)

    vals = (nf1_loss, nf1n_loss, nf2_loss, nf3_loss, nf4_loss, dis_loss,
            role_inc, role_chain,
            nf1_reg, nf1n_reg, nf2_reg, nf3_reg, nf4_reg, dis_reg)
    for o, v in zip(outs, vals):
        o[0, 0] += v


def _tc_compute(big, small):
    scalar_spec = pl.BlockSpec((1, 1), lambda i: (0, 0),
                               memory_space=pltpu.SMEM)
    return pl.pallas_call(
        _tc_body,
        grid=(_B // _CB,),
        in_specs=[
            pl.BlockSpec((_NBIG, _CB, 2 * _DIM), lambda i: (0, i, 0)),
            pl.BlockSpec((_NSMALL, _CB, 2 * _DIM), lambda i: (0, i, 0)),
        ],
        out_specs=[scalar_spec] * 14,
        out_shape=[jax.ShapeDtypeStruct((1, 1), jnp.float32)] * 14,
    )(big, small)


def kernel(min_embedding, delta_embedding, relation_embedding,
           scaling_embedding, data0, data1, data2, data3, data4, data5,
           data6, data7):
    concept_tab = jnp.concatenate([min_embedding, delta_embedding], axis=1)
    relation_tab = jnp.concatenate(
        [relation_embedding, scaling_embedding], axis=1)
    d0 = data0.astype(jnp.int32)
    d1 = data1.astype(jnp.int32)
    d2 = data2.astype(jnp.int32)
    d3 = data3.astype(jnp.int32)
    d4 = data4.astype(jnp.int32)
    d5 = data5.astype(jnp.int32)
    d6 = data6.astype(jnp.int32)
    d7 = data7.astype(jnp.int32)
    idx_big = jnp.stack([
        d0[:, 0], d0[:, 2],
        d1[:, 0], d1[:, 1], d1[:, 2],
        d2[:, 0], d2[:, 2],
        d3[:, 1], d3[:, 2],
        d4[:, 0], d4[:, 1],
        d5[:, 0], d5[:, 2],
    ])
    idx_small = jnp.stack([
        d2[:, 1], d3[:, 0],
        d6[:, 0], d6[:, 1],
        d7[:, 0], d7[:, 1], d7[:, 2],
    ])
    big, small = _sc_gather(concept_tab, relation_tab, idx_big, idx_small)
    outs = _tc_compute(big, small)
    return tuple(o.reshape(()) for o in outs)


# speedup vs baseline: 1.8764x; 1.0526x over previous
import functools
import math

import jax
import jax.numpy as jnp
from jax import lax
from jax.experimental import pallas as pl
from jax.experimental.pallas import tpu as pltpu
from jax.experimental.pallas import tpu_sc as plsc

_EPS = 1e-8
_DIM = 64
_B = 16384
_NBIG = 13
_NSMALL = 7
_NW = 32
_WPB = _B // _NW
_SUB = 128
_CB = 512
_INV_MEAN = 1.0 / (_B * _DIM)
_LOG_LO = math.log(1e-10)
_LOG_HI = math.log(1e4)


def _sc_gather(concept_tab, relation_tab, idx_big, idx_small, nrows):
    mesh = plsc.VectorSubcoreMesh(core_axis_name="c", subcore_axis_name="s")
    wpb = nrows // _NW

    @functools.partial(
        pl.kernel,
        mesh=mesh,
        compiler_params=pltpu.CompilerParams(use_tc_tiling_on_sc=False),
        out_type=[
            jax.ShapeDtypeStruct((_NBIG, nrows, 2 * _DIM), jnp.float32),
            jax.ShapeDtypeStruct((_NSMALL, nrows, 2 * _DIM), jnp.float32),
        ],
        scratch_types=[
            pltpu.VMEM((wpb,), jnp.int32),
            pltpu.VMEM((wpb, 2 * _DIM), jnp.float32),
            pltpu.SemaphoreType.DMA,
        ],
    )
    def gather_kernel(tab_hbm, rel_hbm, ib_hbm, is_hbm, obig_hbm, osmall_hbm,
                      idx_v, buf, sem):
        wid = lax.axis_index("s") * 2 + lax.axis_index("c")
        base = wid * wpb

        def run(tab, out, idx_hbm, nseg):
            for seg in range(nseg):
                pltpu.sync_copy(idx_hbm.at[seg, pl.ds(base, wpb)], idx_v)
                waits = []
                for s in range(wpb // _SUB):
                    sl = pl.ds(s * _SUB, _SUB)
                    waits.append(pltpu.async_copy(
                        tab.at[idx_v.at[sl]], buf.at[sl], sem))
                for w in waits:
                    w.wait()
                pltpu.sync_copy(buf, out.at[seg, pl.ds(base, wpb)])

        run(tab_hbm, obig_hbm, ib_hbm, _NBIG)
        run(rel_hbm, osmall_hbm, is_hbm, _NSMALL)

    return gather_kernel(concept_tab, relation_tab, idx_big, idx_small)


def _lv(diff):
    sp = jnp.maximum(diff, 0.0) + jnp.log1p(jnp.exp(-jnp.abs(diff)))
    return jnp.clip(jnp.sum(jnp.log(sp), axis=1), _LOG_LO, _LOG_HI)


def _reg_sum(mn, mx):
    return (jnp.sum(jnp.maximum(mx - 1.0 + _EPS, 0.0))
            + jnp.sum(jnp.maximum(-mn - _EPS, 0.0)))


def _tc_body(big_ref, small_ref, *outs):
    i = pl.program_id(0)

    @pl.when(i == 0)
    def _():
        for o in outs:
            o[0, 0] = 0.0

    def box(s):
        row = big_ref[s]
        mn = row[:, :_DIM]
        ex = jnp.exp(row[:, _DIM:])
        return mn, mn + ex, ex

    def rels(s):
        row = small_ref[s]
        return row[:, :_DIM], row[:, _DIM:]

    def inter_diff(mn1, mx1, mn2, mx2):
        return jnp.minimum(mx1, mx2) - jnp.maximum(mn1, mn2)

    mn0, mx0, ex0 = box(0)
    mn1, mx1, _ = box(1)
    nf1_loss = jnp.sum(
        1.0 - jnp.exp(_lv(inter_diff(mn0, mx0, mn1, mx1)) - _lv(ex0)))
    nf1_reg = (_reg_sum(mn0, mx0) + _reg_sum(mn1, mx1)) * _INV_MEAN

    mn2, mx2, _ = box(2)
    mn3, mx3, _ = box(3)
    mn4, mx4, _ = box(4)
    imn = jnp.maximum(mn2, mn3)
    imx = jnp.minimum(mx2, mx3)
    nf2_loss = jnp.sum(
        1.0 - jnp.exp(_lv(inter_diff(imn, imx, mn4, mx4)) - _lv(imx - imn)))
    nf2_reg = (_reg_sum(imn, imx) + _reg_sum(mn2, mx2)
               + _reg_sum(mn3, mx3) + _reg_sum(mn4, mx4)) * _INV_MEAN

    mn5, mx5, ex5 = box(5)
    mn6, mx6, _ = box(6)
    rel, sc = rels(0)
    scp = sc + _EPS
    tmn = mn5 * scp + rel
    tmx = mx5 * scp + rel
    nf3_loss = jnp.sum(
        1.0 - jnp.exp(_lv(inter_diff(tmn, tmx, mn6, mx6)) - _lv(ex5 * scp)))
    nf3_reg = (_reg_sum(tmn, tmx) + _reg_sum(mn5, mx5)
               + _reg_sum(mn6, mx6)) * _INV_MEAN

    mn7, mx7, ex7 = box(7)
    mn8, mx8, _ = box(8)
    rel, sc = rels(1)
    scp = sc + _EPS
    tmn = (mn7 - rel) / scp
    tmx = (mx7 - rel) / scp
    nf4_loss = jnp.sum(
        1.0 - jnp.exp(_lv(inter_diff(tmn, tmx, mn8, mx8)) - _lv(ex7 / scp)))
    nf4_reg = (_reg_sum(tmn, tmx) + _reg_sum(mn7, mx7)
               + _reg_sum(mn8, mx8)) * _INV_MEAN

    mn9, mx9, ex9 = box(9)
    mn10, mx10, ex10 = box(10)
    dis_loss = jnp.sum(jnp.exp(
        _lv(inter_diff(mn9, mx9, mn10, mx10)) - (_lv(ex9) + _lv(ex10))))
    dis_reg = (_reg_sum(mn9, mx9) + _reg_sum(mn10, mx10)) * _INV_MEAN

    mn11, mx11, ex11 = box(11)
    mn12, mx12, _ = box(12)
    nf1n_loss = jnp.sum(jnp.exp(
        _lv(inter_diff(mn11, mx11, mn12, mx12)) - _lv(ex11)))
    nf1n_reg = (_reg_sum(mn11, mx11) + _reg_sum(mn12, mx12)) * _INV_MEAN

    t1, s1 = rels(2)
    t2, s2 = rels(3)
    n1 = jnp.sqrt(jnp.sum(jnp.maximum(t1 - t2, 0.0) ** 2, axis=1))
    n2 = jnp.sqrt(jnp.sum(
        jnp.maximum(s1 / (s2 + _EPS) - 1.0, 0.0) ** 2, axis=1))
    role_inc = jnp.sum(n1 + n2)

    t1, s1 = rels(4)
    t2, s2 = rels(5)
    t3, s3 = rels(6)
    n1 = jnp.sqrt(jnp.sum(jnp.maximum(t1 + t2 - t3, 0.0) ** 2, axis=1))
    n2 = jnp.sqrt(jnp.sum(
        jnp.maximum(s1 * s2 / (s3 + _EPS) - 1.0, 0.0) ** 2, axis=1))
    role_chain = jnp.sum(n1 + n2)

    vals = (nf1_loss, nf1n_loss, nf2_loss, nf3_loss, nf4_loss, dis_loss,
            role_inc, role_chain,
            nf1_reg, nf1n_reg, nf2_reg, nf3_reg, nf4_reg, dis_reg)
    for o, v in zip(outs, vals):
        o[0, 0] += v


def _tc_compute(big, small, nrows):
    scalar_spec = pl.BlockSpec((1, 1), lambda i: (0, 0),
                               memory_space=pltpu.SMEM)
    return pl.pallas_call(
        _tc_body,
        grid=(nrows // _CB,),
        in_specs=[
            pl.BlockSpec((_NBIG, _CB, 2 * _DIM), lambda i: (0, i, 0)),
            pl.BlockSpec((_NSMALL, _CB, 2 * _DIM), lambda i: (0, i, 0)),
        ],
        out_specs=[scalar_spec] * 14,
        out_shape=[jax.ShapeDtypeStruct((1, 1), jnp.float32)] * 14,
    )(big, small)


def kernel(min_embedding, delta_embedding, relation_embedding,
           scaling_embedding, data0, data1, data2, data3, data4, data5,
           data6, data7):
    concept_tab = jnp.concatenate([min_embedding, delta_embedding], axis=1)
    relation_tab = jnp.concatenate(
        [relation_embedding, scaling_embedding], axis=1)
    d0 = data0.astype(jnp.int32)
    d1 = data1.astype(jnp.int32)
    d2 = data2.astype(jnp.int32)
    d3 = data3.astype(jnp.int32)
    d4 = data4.astype(jnp.int32)
    d5 = data5.astype(jnp.int32)
    d6 = data6.astype(jnp.int32)
    d7 = data7.astype(jnp.int32)
    idx_big = jnp.stack([
        d0[:, 0], d0[:, 2],
        d1[:, 0], d1[:, 1], d1[:, 2],
        d2[:, 0], d2[:, 2],
        d3[:, 1], d3[:, 2],
        d4[:, 0], d4[:, 1],
        d5[:, 0], d5[:, 2],
    ])
    idx_small = jnp.stack([
        d2[:, 1], d3[:, 0],
        d6[:, 0], d6[:, 1],
        d7[:, 0], d7[:, 1], d7[:, 2],
    ])
    nchunks = 2
    rows = _B // nchunks
    partials = []
    for c in range(nchunks):
        sl = slice(c * rows, (c + 1) * rows)
        big, small = _sc_gather(concept_tab, relation_tab,
                                idx_big[:, sl], idx_small[:, sl], rows)
        partials.append(_tc_compute(big, small, rows))
    outs = [sum(p[i] for p in partials) for i in range(14)]
    return tuple(o.reshape(()) for o in outs)
